# baseline (speedup 1.0000x reference)
import functools

import jax
import jax.numpy as jnp
from jax import lax
from jax.experimental import pallas as pl
from jax.experimental.pallas import tpu as pltpu
from jax.experimental.pallas import tpu_sc as plsc

B, S, F, U, K = 64, 4096, 128, 64, 8
NC, NS = 2, 16
NW = NC * NS
RPW = B // NW
LANES = 16
CHUNKS = S // LANES


SB = 512
NSB = S // SB


def _score_body(x_ref, w1_ref, b1_ref, w2_ref, b2_ref, ones_ref, zero_ref,
                sc_ref):
    x = x_ref[0]
    h = jnp.dot(x, w1_ref[...], preferred_element_type=jnp.float32)
    h = jnp.maximum(h + b1_ref[...], 0.0)
    s = lax.dot_general(w2_ref[...], h, (((0,), (1,)), ((), ())),
                        preferred_element_type=jnp.float32)
    s = s + b2_ref[0, 0]
    nz = (x != 0.0).astype(jnp.float32)
    cnt = lax.dot_general(ones_ref[...], nz, (((0,), (1,)), ((), ())),
                          preferred_element_type=jnp.float32)
    s = jnp.where(cnt > 0.0, s, -1e9)
    sc_ref[...] = s.reshape(1, 1, SB)
    zero_ref[...] = jnp.zeros((SB, F), jnp.float32)


@functools.cache
def _make_score_call():
  return pl.pallas_call(
    _score_body,
    grid=(B, NSB),
    in_specs=[
        pl.BlockSpec((1, SB, F), lambda b, s: (b, s, 0)),
        pl.BlockSpec((F, U), lambda b, s: (0, 0)),
        pl.BlockSpec((1, U), lambda b, s: (0, 0)),
        pl.BlockSpec((U, 1), lambda b, s: (0, 0)),
        pl.BlockSpec((1, 1), lambda b, s: (0, 0), memory_space=pltpu.SMEM),
        pl.BlockSpec((F, 1), lambda b, s: (0, 0)),
    ],
    out_specs=[
        pl.BlockSpec((SB, F), lambda b, s: (b * NSB + s, 0)),
        pl.BlockSpec((1, 1, SB), lambda b, s: (b, 0, s)),
    ],
    out_shape=[
        jax.ShapeDtypeStruct((B * S, F), jnp.float32),
        jax.ShapeDtypeStruct((B, 1, S), jnp.float32),
    ],
  )

def _sc_select_body(scores_hbm, x_hbm, out_hbm, srow, idxv, rows, sem):
    wid = lax.axis_index("s") * NC + lax.axis_index("c")
    lane = lax.iota(jnp.int32, LANES)
    gidx = jnp.zeros((LANES,), jnp.int32)
    for r in range(RPW):
        row = wid * RPW + r
        pltpu.sync_copy(scores_hbm.at[row], srow)
        for k in range(K):
            def body(c, carry):
                bv, bi = carry
                v = srow[pl.ds(c * LANES, LANES)]
                gt = v > bv
                bi = jnp.where(gt, lane + c * LANES, bi)
                bv = jnp.where(gt, v, bv)
                return bv, bi
            bv, bi = lax.fori_loop(
                0, CHUNKS, body,
                (jnp.full((LANES,), -jnp.inf, jnp.float32),
                 jnp.zeros((LANES,), jnp.int32)),
                unroll=8,
            )
            mv = bv[0]
            mi = bi[0]
            for l in range(1, LANES):
                vl = bv[l]
                il = bi[l]
                better = jnp.logical_or(
                    vl > mv, jnp.logical_and(vl == mv, il < mi)
                )
                mv = jnp.where(better, vl, mv)
                mi = jnp.where(better, il, mi)
            gidx = jnp.where(lane == (r * K + k), row * S + mi, gidx)
            base = (mi // LANES) * LANES
            v16 = srow[pl.ds(base, LANES)]
            srow[pl.ds(base, LANES)] = jnp.where(
                lane == mi - base, -jnp.inf, v16
            )
    idxv[0] = gidx
    pltpu.async_copy(x_hbm.at[idxv.at[0]], rows, sem).wait()
    pltpu.async_copy(rows, out_hbm.at[idxv.at[0]], sem).wait()


@functools.cache
def _make_sc_select():
    mesh = plsc.VectorSubcoreMesh(
        core_axis_name="c", subcore_axis_name="s",
        num_cores=NC, num_subcores=NS,
    )
    return pl.kernel(
        _sc_select_body,
        out_type=(),
        mesh=mesh,
        scratch_types=[
            pltpu.VMEM((S,), jnp.float32),
            pltpu.VMEM((1, LANES), jnp.int32),
            pltpu.VMEM((LANES, F), jnp.float32),
            pltpu.SemaphoreType.DMA,
        ],
    )


def kernel(inputs, W1, b1, W2, b2):
    x2d = inputs.reshape(B * S, F)
    zeros2d, scores = _make_score_call()(
        inputs, W1, b1.reshape(1, U), W2, b2.reshape(1, 1),
        jnp.ones((F, 1), jnp.float32),
    )
    out_ref = jax.new_ref(zeros2d)
    _make_sc_select()(scores.reshape(B, S), x2d, out_ref)
    return out_ref[...].reshape(B, S, F)


# speedup vs baseline: 2.0804x; 2.0804x over previous
import functools

import jax
import jax.numpy as jnp
from jax import lax
from jax.experimental import pallas as pl
from jax.experimental.pallas import tpu as pltpu
from jax.experimental.pallas import tpu_sc as plsc

B, S, F, U, K = 64, 4096, 128, 64, 8
NC, NS = 2, 16
NW = NC * NS
RPW = B // NW
LANES = 16
CHUNKS = S // LANES


SB = 2048
NSB = S // SB


def _score_body(x_ref, w1_ref, b1_ref, w2_ref, b2_ref, ones_ref, zero_ref,
                sc_ref):
    x = x_ref[0]
    h = jnp.dot(x, w1_ref[...], preferred_element_type=jnp.float32)
    h = jnp.maximum(h + b1_ref[...], 0.0)
    s = lax.dot_general(w2_ref[...], h, (((0,), (1,)), ((), ())),
                        preferred_element_type=jnp.float32)
    s = s + b2_ref[0, 0]
    nz = (x != 0.0).astype(jnp.float32)
    cnt = lax.dot_general(ones_ref[...], nz, (((0,), (1,)), ((), ())),
                          preferred_element_type=jnp.float32)
    s = jnp.where(cnt > 0.0, s, -1e9)
    sc_ref[...] = s.reshape(1, 1, SB)
    zero_ref[...] = jnp.zeros((SB, F), jnp.float32)


@functools.cache
def _make_score_call():
  return pl.pallas_call(
    _score_body,
    grid=(B, NSB),
    in_specs=[
        pl.BlockSpec((1, SB, F), lambda b, s: (b, s, 0)),
        pl.BlockSpec((F, U), lambda b, s: (0, 0)),
        pl.BlockSpec((1, U), lambda b, s: (0, 0)),
        pl.BlockSpec((U, 1), lambda b, s: (0, 0)),
        pl.BlockSpec((1, 1), lambda b, s: (0, 0), memory_space=pltpu.SMEM),
        pl.BlockSpec((F, 1), lambda b, s: (0, 0)),
    ],
    out_specs=[
        pl.BlockSpec((SB, F), lambda b, s: (b * NSB + s, 0)),
        pl.BlockSpec((1, 1, SB), lambda b, s: (b, 0, s)),
    ],
    out_shape=[
        jax.ShapeDtypeStruct((B * S, F), jnp.float32),
        jax.ShapeDtypeStruct((B, 1, S), jnp.float32),
    ],
  )

def _sc_select_body(scores_hbm, x_hbm, out_hbm, srow, idxv, rows, sem):
    wid = lax.axis_index("s") * NC + lax.axis_index("c")
    lane = lax.iota(jnp.int32, LANES)
    gidx = jnp.zeros((LANES,), jnp.int32)
    for r in range(RPW):
        row = wid * RPW + r
        pltpu.sync_copy(scores_hbm.at[row], srow)
        for k in range(K):
            def body(c, carry):
                bv, bi = carry
                v = srow[pl.ds(c * LANES, LANES)]
                gt = v > bv
                bi = jnp.where(gt, lane + c * LANES, bi)
                bv = jnp.where(gt, v, bv)
                return bv, bi
            bv, bi = lax.fori_loop(
                0, CHUNKS, body,
                (jnp.full((LANES,), -jnp.inf, jnp.float32),
                 jnp.zeros((LANES,), jnp.int32)),
                unroll=8,
            )
            mv = bv[0]
            mi = bi[0]
            for l in range(1, LANES):
                vl = bv[l]
                il = bi[l]
                better = jnp.logical_or(
                    vl > mv, jnp.logical_and(vl == mv, il < mi)
                )
                mv = jnp.where(better, vl, mv)
                mi = jnp.where(better, il, mi)
            gidx = jnp.where(lane == (r * K + k), row * S + mi, gidx)
            base = (mi // LANES) * LANES
            v16 = srow[pl.ds(base, LANES)]
            srow[pl.ds(base, LANES)] = jnp.where(
                lane == mi - base, -jnp.inf, v16
            )
    idxv[0] = gidx
    pltpu.async_copy(x_hbm.at[idxv.at[0]], rows, sem).wait()
    pltpu.async_copy(rows, out_hbm.at[idxv.at[0]], sem).wait()


@functools.cache
def _make_sc_select():
    mesh = plsc.VectorSubcoreMesh(
        core_axis_name="c", subcore_axis_name="s",
        num_cores=NC, num_subcores=NS,
    )
    return pl.kernel(
        _sc_select_body,
        out_type=(),
        mesh=mesh,
        scratch_types=[
            pltpu.VMEM((S,), jnp.float32),
            pltpu.VMEM((1, LANES), jnp.int32),
            pltpu.VMEM((LANES, F), jnp.float32),
            pltpu.SemaphoreType.DMA,
        ],
    )


def kernel(inputs, W1, b1, W2, b2):
    x2d = inputs.reshape(B * S, F)
    zeros2d, scores = _make_score_call()(
        inputs, W1, b1.reshape(1, U), W2, b2.reshape(1, 1),
        jnp.ones((F, 1), jnp.float32),
    )
    out_ref = jax.new_ref(zeros2d)
    _make_sc_select()(scores.reshape(B, S), x2d, out_ref)
    return out_ref[...].reshape(B, S, F)


# speedup vs baseline: 2.7243x; 1.3095x over previous
import functools

import jax
import jax.numpy as jnp
from jax import lax
from jax.experimental import pallas as pl
from jax.experimental.pallas import tpu as pltpu
from jax.experimental.pallas import tpu_sc as plsc

B, S, F, U, K = 64, 4096, 128, 64, 8
NC, NS = 2, 16
NW = NC * NS
RPW = B // NW
LANES = 16
CHUNKS = S // LANES


SB = 4096
NSB = S // SB


def _score_body(x_ref, w1_ref, b1_ref, w2_ref, b2_ref, ones_ref, zero_ref,
                sc_ref):
    x = x_ref[0]
    h = jnp.dot(x, w1_ref[...], preferred_element_type=jnp.float32)
    h = jnp.maximum(h + b1_ref[...], 0.0)
    s = lax.dot_general(w2_ref[...], h, (((0,), (1,)), ((), ())),
                        preferred_element_type=jnp.float32)
    s = s + b2_ref[0, 0]
    nz = (x != 0.0).astype(jnp.float32)
    cnt = lax.dot_general(ones_ref[...], nz, (((0,), (1,)), ((), ())),
                          preferred_element_type=jnp.float32)
    s = jnp.where(cnt > 0.0, s, -1e9)
    sc_ref[...] = s.reshape(1, 1, SB)
    zero_ref[...] = jnp.zeros((SB, F), jnp.float32)


@functools.cache
def _make_score_call():
  return pl.pallas_call(
    _score_body,
    grid=(B, NSB),
    in_specs=[
        pl.BlockSpec((1, SB, F), lambda b, s: (b, s, 0)),
        pl.BlockSpec((F, U), lambda b, s: (0, 0)),
        pl.BlockSpec((1, U), lambda b, s: (0, 0)),
        pl.BlockSpec((U, 1), lambda b, s: (0, 0)),
        pl.BlockSpec((1, 1), lambda b, s: (0, 0), memory_space=pltpu.SMEM),
        pl.BlockSpec((F, 1), lambda b, s: (0, 0)),
    ],
    out_specs=[
        pl.BlockSpec((SB, F), lambda b, s: (b * NSB + s, 0)),
        pl.BlockSpec((1, 1, SB), lambda b, s: (b, 0, s)),
    ],
    out_shape=[
        jax.ShapeDtypeStruct((B * S, F), jnp.float32),
        jax.ShapeDtypeStruct((B, 1, S), jnp.float32),
    ],
  )

def _sc_select_body(scores_hbm, x_hbm, out_hbm, srow, idxv, rows, sem):
    wid = lax.axis_index("s") * NC + lax.axis_index("c")
    lane = lax.iota(jnp.int32, LANES)
    gidx = jnp.zeros((LANES,), jnp.int32)
    for r in range(RPW):
        row = wid * RPW + r
        pltpu.sync_copy(scores_hbm.at[row], srow)
        for k in range(K):
            def body(c, carry):
                bv, bi = carry
                v = srow[pl.ds(c * LANES, LANES)]
                gt = v > bv
                bi = jnp.where(gt, lane + c * LANES, bi)
                bv = jnp.where(gt, v, bv)
                return bv, bi
            bv, bi = lax.fori_loop(
                0, CHUNKS, body,
                (jnp.full((LANES,), -jnp.inf, jnp.float32),
                 jnp.zeros((LANES,), jnp.int32)),
                unroll=8,
            )
            mv = bv[0]
            mi = bi[0]
            for l in range(1, LANES):
                vl = bv[l]
                il = bi[l]
                better = jnp.logical_or(
                    vl > mv, jnp.logical_and(vl == mv, il < mi)
                )
                mv = jnp.where(better, vl, mv)
                mi = jnp.where(better, il, mi)
            gidx = jnp.where(lane == (r * K + k), row * S + mi, gidx)
            base = (mi // LANES) * LANES
            v16 = srow[pl.ds(base, LANES)]
            srow[pl.ds(base, LANES)] = jnp.where(
                lane == mi - base, -jnp.inf, v16
            )
    idxv[0] = gidx
    pltpu.async_copy(x_hbm.at[idxv.at[0]], rows, sem).wait()
    pltpu.async_copy(rows, out_hbm.at[idxv.at[0]], sem).wait()


@functools.cache
def _make_sc_select():
    mesh = plsc.VectorSubcoreMesh(
        core_axis_name="c", subcore_axis_name="s",
        num_cores=NC, num_subcores=NS,
    )
    return pl.kernel(
        _sc_select_body,
        out_type=(),
        mesh=mesh,
        scratch_types=[
            pltpu.VMEM((S,), jnp.float32),
            pltpu.VMEM((1, LANES), jnp.int32),
            pltpu.VMEM((LANES, F), jnp.float32),
            pltpu.SemaphoreType.DMA,
        ],
    )


def kernel(inputs, W1, b1, W2, b2):
    x2d = inputs.reshape(B * S, F)
    zeros2d, scores = _make_score_call()(
        inputs, W1, b1.reshape(1, U), W2, b2.reshape(1, 1),
        jnp.ones((F, 1), jnp.float32),
    )
    out_ref = jax.new_ref(zeros2d)
    _make_sc_select()(scores.reshape(B, S), x2d, out_ref)
    return out_ref[...].reshape(B, S, F)
